# baseline (speedup 1.0000x reference)
import functools

import jax
import jax.numpy as jnp
from jax import lax
from jax.experimental import pallas as pl
from jax.experimental.pallas import tpu as pltpu
from jax.experimental.pallas import tpu_sc as plsc

V = 1000000
D = 32
B = 16384

_info = plsc.get_sparse_core_info()
_NC, _NS = _info.num_cores, _info.num_subcores
_NW = _NC * _NS
_BPW = B // _NW
_CHUNK = 128
_NCHUNK = _BPW // _CHUNK

_mesh = plsc.VectorSubcoreMesh(core_axis_name="c", subcore_axis_name="s")


@functools.partial(
    pl.kernel,
    mesh=_mesh,
    out_type=[
        jax.ShapeDtypeStruct((B, D), jnp.float32),
        jax.ShapeDtypeStruct((B, D), jnp.float32),
        jax.ShapeDtypeStruct((B, D), jnp.float32),
    ],
    scratch_types=[
        pltpu.VMEM((_NCHUNK, _CHUNK), jnp.int32),
        pltpu.VMEM((_BPW, D), jnp.float32),
        pltpu.VMEM((_BPW, D), jnp.float32),
        pltpu.VMEM((_BPW, D), jnp.float32),
        pltpu.SemaphoreType.DMA,
        pltpu.SemaphoreType.DMA,
        pltpu.SemaphoreType.DMA,
    ],
    compiler_params=pltpu.CompilerParams(use_tc_tiling_on_sc=False),
)
def _gather3(idx_hbm, wh_hbm, wd_hbm, ww_hbm, oh_hbm, od_hbm, ow_hbm,
             idx_v, rh, rd, rw, sem_h, sem_d, sem_w):
    wid = lax.axis_index("s") * _NC + lax.axis_index("c")
    base = wid * _BPW
    pltpu.sync_copy(idx_hbm.at[wid], idx_v)
    copies = []
    for j in range(_NCHUNK):
        sl = pl.ds(j * _CHUNK, _CHUNK)
        copies.append(pltpu.async_copy(wh_hbm.at[idx_v.at[j]], rh.at[sl], sem_h))
        copies.append(pltpu.async_copy(wd_hbm.at[idx_v.at[j]], rd.at[sl], sem_d))
        copies.append(pltpu.async_copy(ww_hbm.at[idx_v.at[j]], rw.at[sl], sem_w))
    for c in copies:
        c.wait()
    out_sl = pl.ds(base, _BPW)
    pltpu.sync_copy(rh, oh_hbm.at[out_sl])
    pltpu.sync_copy(rd, od_hbm.at[out_sl])
    pltpu.sync_copy(rw, ow_hbm.at[out_sl])


def kernel(index, W_hour, W_day, W_week):
    idx = index.astype(jnp.int32).reshape(_NW, _NCHUNK, _CHUNK)
    out = _gather3(idx, W_hour, W_day, W_week)
    return tuple(out)


# speedup vs baseline: 1.2391x; 1.2391x over previous
import functools

import jax
import jax.numpy as jnp
from jax import lax
from jax.experimental import pallas as pl
from jax.experimental.pallas import tpu as pltpu
from jax.experimental.pallas import tpu_sc as plsc

V = 1000000
D = 32
B = 16384

_info = plsc.get_sparse_core_info()
_NC, _NS = _info.num_cores, _info.num_subcores
_NW = _NC * _NS
_BPW = B // _NW
_NB = 16
_NG = _BPW // _NB

_mesh = plsc.VectorSubcoreMesh(core_axis_name="c", subcore_axis_name="s")


@functools.partial(
    pl.kernel,
    mesh=_mesh,
    out_type=[
        jax.ShapeDtypeStruct((B, D), jnp.float32),
        jax.ShapeDtypeStruct((B, D), jnp.float32),
        jax.ShapeDtypeStruct((B, D), jnp.float32),
    ],
    scratch_types=[
        pltpu.VMEM((1, _BPW), jnp.int32),
        pltpu.VMEM((_NB, 8, D), jnp.float32),
        pltpu.VMEM((_BPW, D), jnp.float32),
        pltpu.SemaphoreType.DMA((_NB,)),
    ],
    compiler_params=pltpu.CompilerParams(needs_layout_passes=False),
)
def _gather3(idx_hbm, wh_hbm, wd_hbm, ww_hbm, oh_hbm, od_hbm, ow_hbm,
             idx_v, bufs, rows, sems):
    wid = lax.axis_index("s") * _NC + lax.axis_index("c")
    base = wid * _BPW
    pltpu.sync_copy(idx_hbm.at[wid], idx_v)
    tabs = (wh_hbm.reshape(V // 8, 8, D),
            wd_hbm.reshape(V // 8, 8, D),
            ww_hbm.reshape(V // 8, 8, D))
    outs = (oh_hbm, od_hbm, ow_hbm)
    lane = lax.iota(jnp.int32, 16)
    out_sl = pl.ds(base, _BPW)

    for t in range(3):
        tab = tabs[t]

        def fire(vec, l):
            pltpu.async_copy(tab.at[vec[l] >> 3], bufs.at[l], sems.at[l])

        def drain_select(vec, i_base, l):
            pltpu.make_async_copy(tab.at[0], bufs.at[l], sems.at[l]).wait()
            s16 = jnp.full((16,), vec[l] & 7, jnp.int32)
            i16 = jnp.full((16,), i_base + l, jnp.int32)
            for half in range(2):
                c16 = lane + 16 * half
                val = plsc.load_gather(bufs.at[l], [s16, c16])
                plsc.store_scatter(rows, [i16, c16], val)

        v0 = idx_v[0, pl.ds(0, _NB)]
        for l in range(_NB):
            fire(v0, l)

        def body(g, vcur):
            gn = jnp.minimum(g + 1, _NG - 1)
            vnext = idx_v[0, pl.ds(gn * _NB, _NB)]
            i_base = g * _NB
            for l in range(_NB):
                drain_select(vcur, i_base, l)

                @pl.when(g < _NG - 1)
                def _():
                    fire(vnext, l)

            return vnext

        lax.fori_loop(0, _NG, body, v0)
        pltpu.sync_copy(rows, outs[t].at[out_sl])


def kernel(index, W_hour, W_day, W_week):
    idx = index.astype(jnp.int32).reshape(_NW, 1, _BPW)
    out = _gather3(idx, W_hour, W_day, W_week)
    return tuple(out)
